# baseline (speedup 1.0000x reference)
import jax
import jax.numpy as jnp
from jax.experimental import pallas as pl


def _largest_divisor(n, cap):
    d = min(cap, n)
    while n % d:
        d -= 1
    return d


def _gcn_kernel(adj_ref, nf_ref, nfi_ref, W_ref, Ws_ref, bsum_ref, out_ref):
    adj = adj_ref[...]
    g = jnp.dot(adj, nf_ref[...], preferred_element_type=jnp.float32)
    norm = jnp.sum(adj, axis=1, keepdims=True)
    out_ref[...] = (
        jnp.dot(g / norm, W_ref[...].T, preferred_element_type=jnp.float32)
        + jnp.dot(nfi_ref[...], Ws_ref[...].T,
                  preferred_element_type=jnp.float32)
        + bsum_ref[...]
    )


def kernel(node_feat, adj, W, b, W_self, b_self):
    B, N, D_in = node_feat.shape
    D_out = W.shape[0]
    nf = node_feat.reshape(N, D_in)
    bsum = (b + b_self).reshape(1, D_out)

    BM = _largest_divisor(N, 200)
    ni = N // BM

    out = pl.pallas_call(
        _gcn_kernel,
        grid=(ni,),
        in_specs=[
            pl.BlockSpec((BM, N), lambda i: (i, 0)),
            pl.BlockSpec((N, D_in), lambda i: (0, 0)),
            pl.BlockSpec((BM, D_in), lambda i: (i, 0)),
            pl.BlockSpec((D_out, D_in), lambda i: (0, 0)),
            pl.BlockSpec((D_out, D_in), lambda i: (0, 0)),
            pl.BlockSpec((1, D_out), lambda i: (0, 0)),
        ],
        out_specs=pl.BlockSpec((BM, D_out), lambda i: (i, 0)),
        out_shape=jax.ShapeDtypeStruct((N, D_out), jnp.float32),
    )(adj, nf, nf, W, W_self, bsum)

    return out.reshape(B, N, D_out)


# speedup vs baseline: 1.0178x; 1.0178x over previous
import jax
import jax.numpy as jnp
from jax.experimental import pallas as pl
from jax.experimental.pallas import tpu as pltpu


def _largest_divisor(n, cap):
    d = min(cap, n)
    while n % d:
        d -= 1
    return d


def _gcn_kernel(adj_ref, nf_ref, nfi_ref, W_ref, Ws_ref, bsum_ref, out_ref):
    adj = adj_ref[...]
    g = jnp.dot(adj, nf_ref[...], preferred_element_type=jnp.float32)
    norm = jnp.sum(adj, axis=1, keepdims=True)
    out_ref[...] = (
        jnp.dot(g / norm, W_ref[...].T, preferred_element_type=jnp.float32)
        + jnp.dot(nfi_ref[...], Ws_ref[...].T,
                  preferred_element_type=jnp.float32)
        + bsum_ref[...]
    )


def kernel(node_feat, adj, W, b, W_self, b_self):
    B, N, D_in = node_feat.shape
    D_out = W.shape[0]
    nf = node_feat.reshape(N, D_in)
    bsum = (b + b_self).reshape(1, D_out)

    BM = _largest_divisor(N, 400)
    ni = N // BM

    out = pl.pallas_call(
        _gcn_kernel,
        grid=(ni,),
        in_specs=[
            pl.BlockSpec((BM, N), lambda i: (i, 0)),
            pl.BlockSpec((N, D_in), lambda i: (0, 0)),
            pl.BlockSpec((BM, D_in), lambda i: (i, 0)),
            pl.BlockSpec((D_out, D_in), lambda i: (0, 0)),
            pl.BlockSpec((D_out, D_in), lambda i: (0, 0)),
            pl.BlockSpec((1, D_out), lambda i: (0, 0)),
        ],
        out_specs=pl.BlockSpec((BM, D_out), lambda i: (i, 0)),
        out_shape=jax.ShapeDtypeStruct((N, D_out), jnp.float32),
        compiler_params=pltpu.CompilerParams(
            dimension_semantics=("parallel",),
            vmem_limit_bytes=64 * 1024 * 1024,
        ),
    )(adj, nf, nf, W, W_self, bsum)

    return out.reshape(B, N, D_out)


# speedup vs baseline: 1.0656x; 1.0470x over previous
import functools

import jax
import jax.numpy as jnp
from jax.experimental import pallas as pl
from jax.experimental.pallas import tpu as pltpu


def _largest_divisor(n, cap):
    d = min(cap, n)
    while n % d:
        d -= 1
    return d


def _gcn_kernel(adj_ref, nf_ref, W_ref, Ws_ref, bsum_ref, out_ref, *, bm):
    i = pl.program_id(0)
    adj = adj_ref[...]
    g = jnp.dot(adj, nf_ref[...], preferred_element_type=jnp.float32)
    norm = jnp.sum(adj, axis=1, keepdims=True)
    nfi = nf_ref[pl.ds(i * bm, bm), :]
    out_ref[...] = (
        jnp.dot(g / norm, W_ref[...].T, preferred_element_type=jnp.float32)
        + jnp.dot(nfi, Ws_ref[...].T, preferred_element_type=jnp.float32)
        + bsum_ref[...]
    )


def kernel(node_feat, adj, W, b, W_self, b_self):
    B, N, D_in = node_feat.shape
    D_out = W.shape[0]
    nf = node_feat.reshape(N, D_in)
    bsum = (b + b_self).reshape(1, D_out)

    BM = _largest_divisor(N, 400)
    ni = N // BM

    out = pl.pallas_call(
        functools.partial(_gcn_kernel, bm=BM),
        grid=(ni,),
        in_specs=[
            pl.BlockSpec((BM, N), lambda i: (i, 0)),
            pl.BlockSpec((N, D_in), lambda i: (0, 0)),
            pl.BlockSpec((D_out, D_in), lambda i: (0, 0)),
            pl.BlockSpec((D_out, D_in), lambda i: (0, 0)),
            pl.BlockSpec((1, D_out), lambda i: (0, 0)),
        ],
        out_specs=pl.BlockSpec((BM, D_out), lambda i: (i, 0)),
        out_shape=jax.ShapeDtypeStruct((N, D_out), jnp.float32),
        compiler_params=pltpu.CompilerParams(
            dimension_semantics=("parallel",),
            vmem_limit_bytes=64 * 1024 * 1024,
        ),
    )(adj, nf, W, W_self, bsum)

    return out.reshape(B, N, D_out)
